# baseline (speedup 1.0000x reference)
import functools

import jax
import jax.numpy as jnp
from jax import lax
from jax.experimental import pallas as pl
from jax.experimental.pallas import tpu as pltpu
from jax.experimental.pallas import tpu_sc as plsc

NC = 2
NS = 16
L = 16
NW = NC * NS

B = 16384
D = 64
BPW = B // NW
CH = 128
NCH = BPW // CH
GROUPS = BPW // L


def _mf_body(uidx_hbm, iidx_hbm, utab_hbm, itab_hbm, ubias_hbm, ibias_hbm,
             out_hbm, uidx_v, iidx_v, urows_v, irows_v, ub_v, ib_v, res_v,
             sem, bsem):
    wid = lax.axis_index("s") * NC + lax.axis_index("c")

    pltpu.sync_copy(uidx_hbm.at[wid], uidx_v)
    pltpu.sync_copy(iidx_hbm.at[wid], iidx_v)

    pend = []
    for c in range(NCH):
        sl = pl.ds(c * CH, CH)
        pend.append(pltpu.async_copy(utab_hbm.at[uidx_v.at[c]], urows_v.at[sl], sem))
        pend.append(pltpu.async_copy(itab_hbm.at[iidx_v.at[c]], irows_v.at[sl], sem))
        pend.append(pltpu.async_copy(ubias_hbm.at[uidx_v.at[c]], ub_v.at[sl], bsem))
        pend.append(pltpu.async_copy(ibias_hbm.at[iidx_v.at[c]], ib_v.at[sl], bsem))
    for p in pend:
        p.wait()

    lane = lax.iota(jnp.int32, L)

    def gbody(g, carry):
        rows = g * L + lane
        acc = ub_v[pl.ds(g * L, L)] + ib_v[pl.ds(g * L, L)]
        for k in range(D):
            kk = jnp.full((L,), k, jnp.int32)
            u = plsc.load_gather(urows_v, [rows, kk])
            v = plsc.load_gather(irows_v, [rows, kk])
            acc = acc + u * v
        res_v[pl.ds(g * L, L)] = acc
        return carry

    lax.fori_loop(0, GROUPS, gbody, 0)

    pltpu.sync_copy(res_v, out_hbm.at[wid])


@jax.jit
def _mf(user_indices, item_indices, user_embedding, item_embedding,
        user_bias, item_bias):
    uidx = user_indices.astype(jnp.int32).reshape(NW, NCH, CH)
    iidx = item_indices.astype(jnp.int32).reshape(NW, NCH, CH)
    ub = user_bias.reshape(-1)
    ib = item_bias.reshape(-1)

    mesh = plsc.VectorSubcoreMesh(core_axis_name="c", subcore_axis_name="s")
    run = pl.kernel(
        _mf_body,
        out_type=jax.ShapeDtypeStruct((NW, BPW), jnp.float32),
        mesh=mesh,
        compiler_params=pltpu.CompilerParams(
            needs_layout_passes=False, use_tc_tiling_on_sc=False),
        scratch_types=[
            pltpu.VMEM((NCH, CH), jnp.int32),
            pltpu.VMEM((NCH, CH), jnp.int32),
            pltpu.VMEM((BPW, D), jnp.float32),
            pltpu.VMEM((BPW, D), jnp.float32),
            pltpu.VMEM((BPW,), jnp.float32),
            pltpu.VMEM((BPW,), jnp.float32),
            pltpu.VMEM((BPW,), jnp.float32),
            pltpu.SemaphoreType.DMA,
            pltpu.SemaphoreType.DMA,
        ],
    )
    out = run(uidx, iidx, user_embedding, item_embedding, ub, ib)
    return out.reshape(B)


def kernel(user_indices, item_indices, user_embedding, item_embedding,
           user_bias, item_bias):
    return _mf(user_indices, item_indices, user_embedding, item_embedding,
               user_bias, item_bias)
